# Initial kernel scaffold; baseline (speedup 1.0000x reference)
import jax
import jax.numpy as jnp
from jax.experimental import pallas as pl


def kernel(X, init_idx):
    raise NotImplementedError("write your pallas kernel here")


# speedup vs baseline: 1.6082x; 1.6082x over previous
import functools

import jax
import jax.numpy as jnp
from jax import lax
from jax.experimental import pallas as pl
from jax.experimental.pallas import tpu as pltpu
from jax.experimental.pallas import tpu_sc as plsc

N_POINTS = 65536
DIM = 32
K = 512
BLOCK = 2048
NUM_BLOCKS = N_POINTS // BLOCK


def _gather_centroids(X, idx):
    info = plsc.get_sparse_core_info()
    num_workers = info.num_cores * info.num_subcores
    rows_per_worker = K // num_workers
    mesh = plsc.VectorSubcoreMesh(core_axis_name="c", subcore_axis_name="s")

    @functools.partial(
        pl.kernel,
        mesh=mesh,
        out_type=jax.ShapeDtypeStruct((K, DIM), jnp.float32),
        scratch_types=[
            pltpu.VMEM((rows_per_worker,), jnp.int32),
            pltpu.VMEM((rows_per_worker, DIM), jnp.float32),
            pltpu.SemaphoreType.DMA,
        ],
        compiler_params=pltpu.CompilerParams(use_tc_tiling_on_sc=False),
    )
    def gather_kernel(x_hbm, idx_hbm, out_hbm, idx_v, rows_v, sem):
        wid = lax.axis_index("s") * info.num_cores + lax.axis_index("c")
        base = wid * rows_per_worker
        pltpu.sync_copy(idx_hbm.at[pl.ds(base, rows_per_worker)], idx_v)
        pltpu.async_copy(x_hbm.at[idx_v], rows_v, sem).wait()
        pltpu.sync_copy(rows_v, out_hbm.at[pl.ds(base, rows_per_worker)])

    return gather_kernel(X, idx)


def _assign_update_body(x_ref, c_ref, out_ref, acc_ref):
    i = pl.program_id(0)

    @pl.when(i == 0)
    def _init():
        acc_ref[...] = jnp.zeros_like(acc_ref)

    x = x_ref[...]
    c = c_ref[...]
    x_aug = jnp.concatenate(
        [x, jnp.ones((x.shape[0], 1), jnp.float32)], axis=1)
    xc = lax.dot_general(
        x, c, (((1,), (1,)), ((), ())),
        preferred_element_type=jnp.float32,
    )
    x2 = jnp.sum(x * x, axis=1, keepdims=True)
    c2_row = lax.dot_general(
        jnp.ones((1, c.shape[1]), jnp.float32), c * c,
        (((1,), (1,)), ((), ())),
        preferred_element_type=jnp.float32,
        precision=lax.Precision.HIGHEST,
    )
    d2 = (x2 - 2.0 * xc) + c2_row
    m = jnp.min(d2, axis=1, keepdims=True)
    iota = lax.broadcasted_iota(jnp.int32, d2.shape, 1).astype(jnp.float32)
    cand = jnp.where(d2 == m, iota, jnp.float32(K))
    label = jnp.min(cand, axis=1, keepdims=True)
    onehot = (iota == label).astype(jnp.float32)
    acc_ref[...] += lax.dot_general(
        onehot, x_aug, (((0,), (0,)), ((), ())),
        preferred_element_type=jnp.float32,
        precision=lax.Precision.HIGHEST,
    )

    @pl.when(i == NUM_BLOCKS - 1)
    def _finish():
        acc = acc_ref[...]
        counts = acc[:, DIM:DIM + 1]
        out_ref[...] = acc[:, :DIM] / jnp.maximum(counts, 1.0)


def kernel(X, init_idx):
    idx = init_idx.astype(jnp.int32)
    C = _gather_centroids(X, idx)
    centroids = pl.pallas_call(
        _assign_update_body,
        grid=(NUM_BLOCKS,),
        in_specs=[
            pl.BlockSpec((BLOCK, DIM), lambda i: (i, 0)),
            pl.BlockSpec((K, DIM), lambda i: (0, 0)),
        ],
        out_specs=pl.BlockSpec((K, DIM), lambda i: (0, 0)),
        out_shape=jax.ShapeDtypeStruct((K, DIM), jnp.float32),
        scratch_shapes=[pltpu.VMEM((K, DIM + 1), jnp.float32)],
    )(X, C)
    return centroids


# speedup vs baseline: 2.3912x; 1.4869x over previous
import functools

import jax
import jax.numpy as jnp
from jax import lax
from jax.experimental import pallas as pl
from jax.experimental.pallas import tpu as pltpu
from jax.experimental.pallas import tpu_sc as plsc

N_POINTS = 65536
DIM = 32
K = 512
BLOCK = 2048
NUM_BLOCKS = N_POINTS // BLOCK


def _gather_centroids(X, idx):
    info = plsc.get_sparse_core_info()
    num_workers = info.num_cores * info.num_subcores
    rows_per_worker = K // num_workers
    mesh = plsc.VectorSubcoreMesh(core_axis_name="c", subcore_axis_name="s")

    @functools.partial(
        pl.kernel,
        mesh=mesh,
        out_type=jax.ShapeDtypeStruct((K, DIM), jnp.float32),
        scratch_types=[
            pltpu.VMEM((rows_per_worker,), jnp.int32),
            pltpu.VMEM((rows_per_worker, DIM), jnp.float32),
            pltpu.SemaphoreType.DMA,
        ],
        compiler_params=pltpu.CompilerParams(use_tc_tiling_on_sc=False),
    )
    def gather_kernel(x_hbm, idx_hbm, out_hbm, idx_v, rows_v, sem):
        wid = lax.axis_index("s") * info.num_cores + lax.axis_index("c")
        base = wid * rows_per_worker
        pltpu.sync_copy(idx_hbm.at[pl.ds(base, rows_per_worker)], idx_v)
        pltpu.async_copy(x_hbm.at[idx_v], rows_v, sem).wait()
        pltpu.sync_copy(rows_v, out_hbm.at[pl.ds(base, rows_per_worker)])

    return gather_kernel(X, idx)


def _assign_update_body(x_ref, c_ref, out_ref, acc_ref):
    i = pl.program_id(0)

    @pl.when(i == 0)
    def _init():
        acc_ref[...] = jnp.zeros_like(acc_ref)

    x = x_ref[...]
    c = c_ref[...]
    x_aug = jnp.concatenate(
        [x, jnp.ones((x.shape[0], 1), jnp.float32)], axis=1)
    xc = lax.dot_general(
        x, c, (((1,), (1,)), ((), ())),
        preferred_element_type=jnp.float32,
    )
    x2 = jnp.sum(x * x, axis=1, keepdims=True)
    c2_row = lax.dot_general(
        jnp.ones((1, c.shape[1]), jnp.float32), c * c,
        (((1,), (1,)), ((), ())),
        preferred_element_type=jnp.float32,
        precision=lax.Precision.HIGHEST,
    )
    d2 = (x2 - 2.0 * xc) + c2_row
    m = jnp.min(d2, axis=1, keepdims=True)
    iota = lax.broadcasted_iota(jnp.int32, d2.shape, 1).astype(jnp.float32)
    cand = jnp.where(d2 == m, iota, jnp.float32(K))
    label = jnp.min(cand, axis=1, keepdims=True)
    onehot = (iota == label).astype(jnp.float32)
    acc_ref[...] += lax.dot_general(
        onehot, x_aug, (((0,), (0,)), ((), ())),
        preferred_element_type=jnp.float32,
    )

    @pl.when(i == NUM_BLOCKS - 1)
    def _finish():
        acc = acc_ref[...]
        counts = acc[:, DIM:DIM + 1]
        out_ref[...] = acc[:, :DIM] / jnp.maximum(counts, 1.0)


def kernel(X, init_idx):
    idx = init_idx.astype(jnp.int32)
    C = _gather_centroids(X, idx)
    centroids = pl.pallas_call(
        _assign_update_body,
        grid=(NUM_BLOCKS,),
        in_specs=[
            pl.BlockSpec((BLOCK, DIM), lambda i: (i, 0)),
            pl.BlockSpec((K, DIM), lambda i: (0, 0)),
        ],
        out_specs=pl.BlockSpec((K, DIM), lambda i: (0, 0)),
        out_shape=jax.ShapeDtypeStruct((K, DIM), jnp.float32),
        scratch_shapes=[pltpu.VMEM((K, DIM + 1), jnp.float32)],
    )(X, C)
    return centroids


# speedup vs baseline: 2.4332x; 1.0175x over previous
import functools

import jax
import jax.numpy as jnp
from jax import lax
from jax.experimental import pallas as pl
from jax.experimental.pallas import tpu as pltpu
from jax.experimental.pallas import tpu_sc as plsc

N_POINTS = 65536
DIM = 32
K = 512
BLOCK = 4096
NUM_BLOCKS = N_POINTS // BLOCK


def _gather_centroids(X, idx):
    info = plsc.get_sparse_core_info()
    num_workers = info.num_cores * info.num_subcores
    rows_per_worker = K // num_workers
    mesh = plsc.VectorSubcoreMesh(core_axis_name="c", subcore_axis_name="s")

    @functools.partial(
        pl.kernel,
        mesh=mesh,
        out_type=jax.ShapeDtypeStruct((K, DIM), jnp.float32),
        scratch_types=[
            pltpu.VMEM((rows_per_worker,), jnp.int32),
            pltpu.VMEM((rows_per_worker, DIM), jnp.float32),
            pltpu.SemaphoreType.DMA,
        ],
        compiler_params=pltpu.CompilerParams(use_tc_tiling_on_sc=False),
    )
    def gather_kernel(x_hbm, idx_hbm, out_hbm, idx_v, rows_v, sem):
        wid = lax.axis_index("s") * info.num_cores + lax.axis_index("c")
        base = wid * rows_per_worker
        pltpu.sync_copy(idx_hbm.at[pl.ds(base, rows_per_worker)], idx_v)
        pltpu.async_copy(x_hbm.at[idx_v], rows_v, sem).wait()
        pltpu.sync_copy(rows_v, out_hbm.at[pl.ds(base, rows_per_worker)])

    return gather_kernel(X, idx)


def _assign_update_body(x_ref, c_ref, out_ref, acc_ref):
    i = pl.program_id(0)

    @pl.when(i == 0)
    def _init():
        acc_ref[...] = jnp.zeros_like(acc_ref)

    x = x_ref[...]
    c = c_ref[...]
    x_aug = jnp.concatenate(
        [x, jnp.ones((x.shape[0], 1), jnp.float32)], axis=1)
    neg2xc = lax.dot_general(
        x, -2.0 * c, (((1,), (1,)), ((), ())),
        preferred_element_type=jnp.float32,
    )
    x2 = jnp.sum(x * x, axis=1, keepdims=True)
    c2_row = lax.dot_general(
        jnp.ones((1, c.shape[1]), jnp.float32), c * c,
        (((1,), (1,)), ((), ())),
        preferred_element_type=jnp.float32,
        precision=lax.Precision.HIGHEST,
    )
    d2 = (x2 + neg2xc) + c2_row
    m = jnp.min(d2, axis=1, keepdims=True)
    iota = lax.broadcasted_iota(jnp.int32, d2.shape, 1).astype(jnp.float32)
    cand = jnp.where(d2 == m, iota, jnp.float32(K))
    label = jnp.min(cand, axis=1, keepdims=True)
    onehot = (iota == label).astype(jnp.float32)
    acc_ref[...] += lax.dot_general(
        onehot, x_aug, (((0,), (0,)), ((), ())),
        preferred_element_type=jnp.float32,
    )

    @pl.when(i == NUM_BLOCKS - 1)
    def _finish():
        acc = acc_ref[...]
        counts = acc[:, DIM:DIM + 1]
        out_ref[...] = acc[:, :DIM] / jnp.maximum(counts, 1.0)


def kernel(X, init_idx):
    idx = init_idx.astype(jnp.int32)
    C = _gather_centroids(X, idx)
    centroids = pl.pallas_call(
        _assign_update_body,
        grid=(NUM_BLOCKS,),
        in_specs=[
            pl.BlockSpec((BLOCK, DIM), lambda i: (i, 0)),
            pl.BlockSpec((K, DIM), lambda i: (0, 0)),
        ],
        out_specs=pl.BlockSpec((K, DIM), lambda i: (0, 0)),
        out_shape=jax.ShapeDtypeStruct((K, DIM), jnp.float32),
        scratch_shapes=[pltpu.VMEM((K, DIM + 1), jnp.float32)],
    )(X, C)
    return centroids


# speedup vs baseline: 3.1625x; 1.2998x over previous
import functools

import jax
import jax.numpy as jnp
from jax import lax
from jax.experimental import pallas as pl
from jax.experimental.pallas import tpu as pltpu
from jax.experimental.pallas import tpu_sc as plsc

N_POINTS = 65536
DIM = 32
K = 512
BLOCK = 4096
NUM_BLOCKS = N_POINTS // BLOCK


def _gather_centroids(X, idx):
    info = plsc.get_sparse_core_info()
    num_workers = info.num_cores * info.num_subcores
    rows_per_worker = K // num_workers
    mesh = plsc.VectorSubcoreMesh(core_axis_name="c", subcore_axis_name="s")

    @functools.partial(
        pl.kernel,
        mesh=mesh,
        out_type=jax.ShapeDtypeStruct((K, DIM), jnp.float32),
        scratch_types=[
            pltpu.VMEM((rows_per_worker,), jnp.int32),
            pltpu.VMEM((rows_per_worker, DIM), jnp.float32),
            pltpu.SemaphoreType.DMA,
        ],
        compiler_params=pltpu.CompilerParams(use_tc_tiling_on_sc=False),
    )
    def gather_kernel(x_hbm, idx_hbm, out_hbm, idx_v, rows_v, sem):
        wid = lax.axis_index("s") * info.num_cores + lax.axis_index("c")
        base = wid * rows_per_worker
        pltpu.sync_copy(idx_hbm.at[pl.ds(base, rows_per_worker)], idx_v)
        pltpu.async_copy(x_hbm.at[idx_v], rows_v, sem).wait()
        pltpu.sync_copy(rows_v, out_hbm.at[pl.ds(base, rows_per_worker)])

    return gather_kernel(X, idx)


def _assign_update_body(x_ref, c_ref, out_ref, acc_ref):
    i = pl.program_id(0)

    @pl.when(i == 0)
    def _init():
        acc_ref[...] = jnp.zeros_like(acc_ref)

    x = x_ref[...]
    c = c_ref[...]
    c2_col = jnp.sum(c * c, axis=1, keepdims=True)
    c2_hi = c2_col.astype(jnp.bfloat16).astype(jnp.float32)
    r1 = c2_col - c2_hi
    c2_mid = r1.astype(jnp.bfloat16).astype(jnp.float32)
    c2_lo = r1 - c2_mid
    c_aug = jnp.concatenate([-2.0 * c, c2_hi, c2_mid, c2_lo], axis=1)
    ones3 = jnp.ones((x.shape[0], 3), jnp.float32)
    d2 = lax.dot_general(
        jnp.concatenate([x, ones3], axis=1), c_aug,
        (((1,), (1,)), ((), ())),
        preferred_element_type=jnp.float32,
    )
    m = jnp.min(d2, axis=1, keepdims=True)
    iota = lax.broadcasted_iota(jnp.int32, d2.shape, 1).astype(jnp.float32)
    cand = jnp.where(d2 == m, iota, jnp.float32(K))
    label = jnp.min(cand, axis=1, keepdims=True)
    onehot = (iota == label).astype(jnp.bfloat16)
    x_aug = jnp.concatenate(
        [x, jnp.ones((x.shape[0], 1), jnp.float32)], axis=1).astype(jnp.bfloat16)
    acc_ref[...] += lax.dot_general(
        onehot, x_aug, (((0,), (0,)), ((), ())),
        preferred_element_type=jnp.float32,
    )

    @pl.when(i == NUM_BLOCKS - 1)
    def _finish():
        acc = acc_ref[...]
        counts = acc[:, DIM:DIM + 1]
        out_ref[...] = acc[:, :DIM] / jnp.maximum(counts, 1.0)


def kernel(X, init_idx):
    idx = init_idx.astype(jnp.int32)
    C = _gather_centroids(X, idx)
    centroids = pl.pallas_call(
        _assign_update_body,
        grid=(NUM_BLOCKS,),
        in_specs=[
            pl.BlockSpec((BLOCK, DIM), lambda i: (i, 0)),
            pl.BlockSpec((K, DIM), lambda i: (0, 0)),
        ],
        out_specs=pl.BlockSpec((K, DIM), lambda i: (0, 0)),
        out_shape=jax.ShapeDtypeStruct((K, DIM), jnp.float32),
        scratch_shapes=[pltpu.VMEM((K, DIM + 1), jnp.float32)],
    )(X, C)
    return centroids


# speedup vs baseline: 3.3446x; 1.0576x over previous
import functools

import jax
import jax.numpy as jnp
from jax import lax
from jax.experimental import pallas as pl
from jax.experimental.pallas import tpu as pltpu
from jax.experimental.pallas import tpu_sc as plsc

N_POINTS = 65536
DIM = 32
K = 512
BLOCK = 8192
NUM_BLOCKS = N_POINTS // BLOCK


def _gather_centroids(X, idx):
    info = plsc.get_sparse_core_info()
    num_workers = info.num_cores * info.num_subcores
    rows_per_worker = K // num_workers
    mesh = plsc.VectorSubcoreMesh(core_axis_name="c", subcore_axis_name="s")

    @functools.partial(
        pl.kernel,
        mesh=mesh,
        out_type=jax.ShapeDtypeStruct((K, DIM), jnp.float32),
        scratch_types=[
            pltpu.VMEM((rows_per_worker,), jnp.int32),
            pltpu.VMEM((rows_per_worker, DIM), jnp.float32),
            pltpu.SemaphoreType.DMA,
        ],
        compiler_params=pltpu.CompilerParams(use_tc_tiling_on_sc=False),
    )
    def gather_kernel(x_hbm, idx_hbm, out_hbm, idx_v, rows_v, sem):
        wid = lax.axis_index("s") * info.num_cores + lax.axis_index("c")
        base = wid * rows_per_worker
        pltpu.sync_copy(idx_hbm.at[pl.ds(base, rows_per_worker)], idx_v)
        pltpu.async_copy(x_hbm.at[idx_v], rows_v, sem).wait()
        pltpu.sync_copy(rows_v, out_hbm.at[pl.ds(base, rows_per_worker)])

    return gather_kernel(X, idx)


def _assign_update_body(x_ref, c_ref, out_ref, acc_ref):
    i = pl.program_id(0)

    @pl.when(i == 0)
    def _init():
        acc_ref[...] = jnp.zeros_like(acc_ref)

    x = x_ref[...]
    c = c_ref[...]
    c2_col = jnp.sum(c * c, axis=1, keepdims=True)
    c2_hi = c2_col.astype(jnp.bfloat16).astype(jnp.float32)
    r1 = c2_col - c2_hi
    c2_mid = r1.astype(jnp.bfloat16).astype(jnp.float32)
    c2_lo = r1 - c2_mid
    c_aug = jnp.concatenate([-2.0 * c, c2_hi, c2_mid, c2_lo], axis=1)
    ones3 = jnp.ones((x.shape[0], 3), jnp.float32)
    d2 = lax.dot_general(
        jnp.concatenate([x, ones3], axis=1), c_aug,
        (((1,), (1,)), ((), ())),
        preferred_element_type=jnp.float32,
    )
    m = jnp.min(d2, axis=1, keepdims=True)
    iota = lax.broadcasted_iota(jnp.int32, d2.shape, 1).astype(jnp.float32)
    cand = jnp.where(d2 == m, iota, jnp.float32(K))
    label = jnp.min(cand, axis=1, keepdims=True)
    onehot = (iota == label).astype(jnp.bfloat16)
    x_aug = jnp.concatenate(
        [x, jnp.ones((x.shape[0], 1), jnp.float32)], axis=1).astype(jnp.bfloat16)
    acc_ref[...] += lax.dot_general(
        onehot, x_aug, (((0,), (0,)), ((), ())),
        preferred_element_type=jnp.float32,
    )

    @pl.when(i == NUM_BLOCKS - 1)
    def _finish():
        acc = acc_ref[...]
        counts = acc[:, DIM:DIM + 1]
        out_ref[...] = acc[:, :DIM] / jnp.maximum(counts, 1.0)


def kernel(X, init_idx):
    idx = init_idx.astype(jnp.int32)
    C = _gather_centroids(X, idx)
    centroids = pl.pallas_call(
        _assign_update_body,
        grid=(NUM_BLOCKS,),
        in_specs=[
            pl.BlockSpec((BLOCK, DIM), lambda i: (i, 0)),
            pl.BlockSpec((K, DIM), lambda i: (0, 0)),
        ],
        out_specs=pl.BlockSpec((K, DIM), lambda i: (0, 0)),
        out_shape=jax.ShapeDtypeStruct((K, DIM), jnp.float32),
        scratch_shapes=[pltpu.VMEM((K, DIM + 1), jnp.float32)],
    )(X, C)
    return centroids


# speedup vs baseline: 4.1307x; 1.2350x over previous
import jax
import jax.numpy as jnp
from jax import lax
from jax.experimental import pallas as pl
from jax.experimental.pallas import tpu as pltpu

N_POINTS = 65536
DIM = 32
K = 512
BL = 4096
NB = N_POINTS // BL


def _split3(x):
    hi = x.astype(jnp.bfloat16)
    r1 = x - hi.astype(jnp.float32)
    mid = r1.astype(jnp.bfloat16)
    lo = (r1 - mid.astype(jnp.float32)).astype(jnp.bfloat16)
    return hi, mid, lo


def _extract_body(xt_ref, idx_ref, ct_ref, acc_ref):
    i = pl.program_id(0)

    @pl.when(i == 0)
    def _init():
        acc_ref[...] = jnp.zeros_like(acc_ref)

    xt = xt_ref[...]
    idx_row = idx_ref[...]
    iota0 = lax.broadcasted_iota(jnp.int32, (BL, K), 0) + i * BL
    sel = (iota0 == idx_row).astype(jnp.bfloat16)
    for chunk in _split3(xt):
        acc_ref[...] += lax.dot_general(
            chunk, sel, (((1,), (0,)), ((), ())),
            preferred_element_type=jnp.float32,
        )

    @pl.when(i == NB - 1)
    def _finish():
        ct_ref[...] = acc_ref[...]


def _assign_body(xt_ref, ct_ref, out_ref, caug_ref, acc_ref):
    i = pl.program_id(0)

    @pl.when(i == 0)
    def _prep():
        acc_ref[...] = jnp.zeros_like(acc_ref)
        ct = ct_ref[...]
        eye = (lax.broadcasted_iota(jnp.int32, (DIM, DIM), 0)
               == lax.broadcasted_iota(jnp.int32, (DIM, DIM), 1)
               ).astype(jnp.float32)
        c = lax.dot_general(
            ct, eye, (((0,), (0,)), ((), ())),
            preferred_element_type=jnp.float32,
            precision=lax.Precision.HIGHEST,
        )
        c2 = jnp.sum(c * c, axis=1, keepdims=True)
        c2_hi = c2.astype(jnp.bfloat16).astype(jnp.float32)
        r1 = c2 - c2_hi
        c2_mid = r1.astype(jnp.bfloat16).astype(jnp.float32)
        c2_lo = r1 - c2_mid
        caug_ref[...] = jnp.concatenate(
            [-2.0 * c, c2_hi, c2_mid, c2_lo], axis=1)

    xt = xt_ref[...]
    x_aug = jnp.concatenate(
        [xt, jnp.ones((3, BL), jnp.float32)], axis=0)
    d2 = lax.dot_general(
        caug_ref[...], x_aug, (((1,), (0,)), ((), ())),
        preferred_element_type=jnp.float32,
    )
    m = jnp.min(d2, axis=0, keepdims=True)
    iota0 = lax.broadcasted_iota(jnp.int32, (K, BL), 0).astype(jnp.float32)
    cand = jnp.where(d2 == m, iota0, jnp.float32(K))
    label = jnp.min(cand, axis=0, keepdims=True)
    onehot = (iota0 == label).astype(jnp.bfloat16)
    x1 = jnp.concatenate(
        [xt, jnp.ones((1, BL), jnp.float32)], axis=0).astype(jnp.bfloat16)
    acc_ref[...] += lax.dot_general(
        onehot, x1, (((1,), (1,)), ((), ())),
        preferred_element_type=jnp.float32,
    )

    @pl.when(i == NB - 1)
    def _finish():
        acc = acc_ref[...]
        counts = acc[:, DIM:DIM + 1]
        out_ref[...] = acc[:, :DIM] / jnp.maximum(counts, 1.0)


def kernel(X, init_idx):
    idx_row = init_idx.astype(jnp.int32).reshape(1, K)
    Xt = X.T
    ct = pl.pallas_call(
        _extract_body,
        grid=(NB,),
        in_specs=[
            pl.BlockSpec((DIM, BL), lambda i: (0, i)),
            pl.BlockSpec((1, K), lambda i: (0, 0)),
        ],
        out_specs=pl.BlockSpec((DIM, K), lambda i: (0, 0)),
        out_shape=jax.ShapeDtypeStruct((DIM, K), jnp.float32),
        scratch_shapes=[pltpu.VMEM((DIM, K), jnp.float32)],
    )(Xt, idx_row)
    centroids = pl.pallas_call(
        _assign_body,
        grid=(NB,),
        in_specs=[
            pl.BlockSpec((DIM, BL), lambda i: (0, i)),
            pl.BlockSpec((DIM, K), lambda i: (0, 0)),
        ],
        out_specs=pl.BlockSpec((K, DIM), lambda i: (0, 0)),
        out_shape=jax.ShapeDtypeStruct((K, DIM), jnp.float32),
        scratch_shapes=[
            pltpu.VMEM((K, DIM + 3), jnp.float32),
            pltpu.VMEM((K, DIM + 1), jnp.float32),
        ],
    )(Xt, ct)
    return centroids


# speedup vs baseline: 4.1743x; 1.0106x over previous
import jax
import jax.numpy as jnp
from jax import lax
from jax.experimental import pallas as pl
from jax.experimental.pallas import tpu as pltpu

N_POINTS = 65536
DIM = 32
K = 512
BL = 8192
NB = N_POINTS // BL


def _split3(x):
    hi = x.astype(jnp.bfloat16)
    r1 = x - hi.astype(jnp.float32)
    mid = r1.astype(jnp.bfloat16)
    lo = (r1 - mid.astype(jnp.float32)).astype(jnp.bfloat16)
    return hi, mid, lo


def _extract_body(xt_ref, idx_ref, ct_ref, acc_ref):
    i = pl.program_id(0)

    @pl.when(i == 0)
    def _init():
        acc_ref[...] = jnp.zeros_like(acc_ref)

    xt = xt_ref[...]
    idx_row = idx_ref[...]
    iota0 = lax.broadcasted_iota(jnp.int32, (BL, K), 0) + i * BL
    sel = (iota0 == idx_row).astype(jnp.bfloat16)
    for chunk in _split3(xt):
        acc_ref[...] += lax.dot_general(
            chunk, sel, (((1,), (0,)), ((), ())),
            preferred_element_type=jnp.float32,
        )

    @pl.when(i == NB - 1)
    def _finish():
        ct_ref[...] = acc_ref[...]


def _assign_body(xt_ref, ct_ref, out_ref, caug_ref, acc_ref):
    i = pl.program_id(0)

    @pl.when(i == 0)
    def _prep():
        acc_ref[...] = jnp.zeros_like(acc_ref)
        ct = ct_ref[...]
        eye = (lax.broadcasted_iota(jnp.int32, (DIM, DIM), 0)
               == lax.broadcasted_iota(jnp.int32, (DIM, DIM), 1)
               ).astype(jnp.float32)
        c = lax.dot_general(
            ct, eye, (((0,), (0,)), ((), ())),
            preferred_element_type=jnp.float32,
            precision=lax.Precision.HIGHEST,
        )
        c2 = jnp.sum(c * c, axis=1, keepdims=True)
        c2_hi = c2.astype(jnp.bfloat16).astype(jnp.float32)
        r1 = c2 - c2_hi
        c2_mid = r1.astype(jnp.bfloat16).astype(jnp.float32)
        c2_lo = r1 - c2_mid
        caug_ref[...] = jnp.concatenate(
            [-2.0 * c, c2_hi, c2_mid, c2_lo], axis=1)

    xt = xt_ref[...]
    x_aug = jnp.concatenate(
        [xt, jnp.ones((3, BL), jnp.float32)], axis=0)
    d2 = lax.dot_general(
        caug_ref[...], x_aug, (((1,), (0,)), ((), ())),
        preferred_element_type=jnp.float32,
    )
    m = jnp.min(d2, axis=0, keepdims=True)
    iota0 = lax.broadcasted_iota(jnp.int32, (K, BL), 0).astype(jnp.float32)
    cand = jnp.where(d2 == m, iota0, jnp.float32(K))
    label = jnp.min(cand, axis=0, keepdims=True)
    onehot = (iota0 == label).astype(jnp.bfloat16)
    x1 = jnp.concatenate(
        [xt, jnp.ones((1, BL), jnp.float32)], axis=0).astype(jnp.bfloat16)
    acc_ref[...] += lax.dot_general(
        onehot, x1, (((1,), (1,)), ((), ())),
        preferred_element_type=jnp.float32,
    )

    @pl.when(i == NB - 1)
    def _finish():
        acc = acc_ref[...]
        counts = acc[:, DIM:DIM + 1]
        out_ref[...] = acc[:, :DIM] / jnp.maximum(counts, 1.0)


def kernel(X, init_idx):
    idx_row = init_idx.astype(jnp.int32).reshape(1, K)
    Xt = X.T
    ct = pl.pallas_call(
        _extract_body,
        grid=(NB,),
        in_specs=[
            pl.BlockSpec((DIM, BL), lambda i: (0, i)),
            pl.BlockSpec((1, K), lambda i: (0, 0)),
        ],
        out_specs=pl.BlockSpec((DIM, K), lambda i: (0, 0)),
        out_shape=jax.ShapeDtypeStruct((DIM, K), jnp.float32),
        scratch_shapes=[pltpu.VMEM((DIM, K), jnp.float32)],
    )(Xt, idx_row)
    centroids = pl.pallas_call(
        _assign_body,
        grid=(NB,),
        in_specs=[
            pl.BlockSpec((DIM, BL), lambda i: (0, i)),
            pl.BlockSpec((DIM, K), lambda i: (0, 0)),
        ],
        out_specs=pl.BlockSpec((K, DIM), lambda i: (0, 0)),
        out_shape=jax.ShapeDtypeStruct((K, DIM), jnp.float32),
        scratch_shapes=[
            pltpu.VMEM((K, DIM + 3), jnp.float32),
            pltpu.VMEM((K, DIM + 1), jnp.float32),
        ],
    )(Xt, ct)
    return centroids


# speedup vs baseline: 4.2291x; 1.0131x over previous
import jax
import jax.numpy as jnp
from jax import lax
from jax.experimental import pallas as pl
from jax.experimental.pallas import tpu as pltpu

N_POINTS = 65536
DIM = 32
K = 512
BL = 8192
NB = N_POINTS // BL


def _split3(x):
    hi = x.astype(jnp.bfloat16)
    r1 = x - hi.astype(jnp.float32)
    mid = r1.astype(jnp.bfloat16)
    lo = (r1 - mid.astype(jnp.float32)).astype(jnp.bfloat16)
    return hi, mid, lo


def _fused_body(xt_ref, idx_ref, out_ref, ct_ref, caug_ref, acc_ref):
    p = pl.program_id(0)
    i = pl.program_id(1)
    xt = xt_ref[...]

    @pl.when((p == 0) & (i == 0))
    def _init():
        ct_ref[...] = jnp.zeros_like(ct_ref)
        acc_ref[...] = jnp.zeros_like(acc_ref)

    @pl.when(p == 0)
    def _extract():
        idx_row = idx_ref[...]
        iota0 = lax.broadcasted_iota(jnp.int32, (BL, K), 0) + i * BL
        sel = (iota0 == idx_row).astype(jnp.bfloat16)
        for chunk in _split3(xt):
            ct_ref[...] += lax.dot_general(
                chunk, sel, (((1,), (0,)), ((), ())),
                preferred_element_type=jnp.float32,
            )

    @pl.when((p == 1) & (i == 0))
    def _prep():
        ct = ct_ref[...]
        eye = (lax.broadcasted_iota(jnp.int32, (DIM, DIM), 0)
               == lax.broadcasted_iota(jnp.int32, (DIM, DIM), 1)
               ).astype(jnp.float32)
        c = lax.dot_general(
            ct, eye, (((0,), (0,)), ((), ())),
            preferred_element_type=jnp.float32,
            precision=lax.Precision.HIGHEST,
        )
        c2 = jnp.sum(c * c, axis=1, keepdims=True)
        c2_hi = c2.astype(jnp.bfloat16).astype(jnp.float32)
        r1 = c2 - c2_hi
        c2_mid = r1.astype(jnp.bfloat16).astype(jnp.float32)
        c2_lo = r1 - c2_mid
        caug_ref[...] = jnp.concatenate(
            [-2.0 * c, c2_hi, c2_mid, c2_lo], axis=1)

    @pl.when(p == 1)
    def _assign():
        x_aug = jnp.concatenate(
            [xt, jnp.ones((3, BL), jnp.float32)], axis=0)
        d2 = lax.dot_general(
            caug_ref[...], x_aug, (((1,), (0,)), ((), ())),
            preferred_element_type=jnp.float32,
        )
        m = jnp.min(d2, axis=0, keepdims=True)
        iota0 = lax.broadcasted_iota(jnp.int32, (K, BL), 0).astype(jnp.float32)
        cand = jnp.where(d2 == m, iota0, jnp.float32(K))
        label = jnp.min(cand, axis=0, keepdims=True)
        onehot = (iota0 == label).astype(jnp.bfloat16)
        x1 = jnp.concatenate(
            [xt, jnp.ones((1, BL), jnp.float32)], axis=0).astype(jnp.bfloat16)
        acc_ref[...] += lax.dot_general(
            onehot, x1, (((1,), (1,)), ((), ())),
            preferred_element_type=jnp.float32,
        )

    @pl.when((p == 1) & (i == NB - 1))
    def _finish():
        acc = acc_ref[...]
        counts = acc[:, DIM:DIM + 1]
        out_ref[...] = acc[:, :DIM] / jnp.maximum(counts, 1.0)


def kernel(X, init_idx):
    idx_row = init_idx.astype(jnp.int32).reshape(1, K)
    Xt = X.T
    centroids = pl.pallas_call(
        _fused_body,
        grid=(2, NB),
        in_specs=[
            pl.BlockSpec((DIM, BL), lambda p, i: (0, i)),
            pl.BlockSpec((1, K), lambda p, i: (0, 0)),
        ],
        out_specs=pl.BlockSpec((K, DIM), lambda p, i: (0, 0)),
        out_shape=jax.ShapeDtypeStruct((K, DIM), jnp.float32),
        scratch_shapes=[
            pltpu.VMEM((DIM, K), jnp.float32),
            pltpu.VMEM((K, DIM + 3), jnp.float32),
            pltpu.VMEM((K, DIM + 1), jnp.float32),
        ],
    )(Xt, idx_row)
    return centroids
